# baseline (speedup 1.0000x reference)
import functools

import jax
import jax.numpy as jnp
from jax import lax
from jax.experimental import pallas as pl
from jax.experimental.pallas import tpu as pltpu
from jax.experimental.pallas import tpu_sc as plsc

EMBED_DIM = 64
BATCH = 16384

_info = plsc.get_sparse_core_info()
_NC, _NS, _L = _info.num_cores, _info.num_subcores, _info.num_lanes
_NW = _NC * _NS
_BPW = BATCH // _NW
_CHUNK = 128
_NCHUNK = _BPW // _CHUNK

_mesh = plsc.VectorSubcoreMesh(core_axis_name="c", subcore_axis_name="s")


@functools.partial(
    pl.kernel,
    mesh=_mesh,
    out_type=jax.ShapeDtypeStruct((BATCH,), jnp.float32),
    scratch_types=[
        pltpu.VMEM((_BPW,), jnp.int32),
        pltpu.VMEM((_BPW,), jnp.int32),
        pltpu.VMEM((_BPW,), jnp.int32),
        pltpu.VMEM((_BPW, EMBED_DIM), jnp.float32),
        pltpu.VMEM((_BPW, EMBED_DIM), jnp.float32),
        pltpu.VMEM((_BPW, EMBED_DIM), jnp.float32),
        pltpu.VMEM((_BPW,), jnp.float32),
        pltpu.SemaphoreType.DMA,
    ],
    compiler_params=pltpu.CompilerParams(needs_layout_passes=False,
                                         use_tc_tiling_on_sc=False),
)
def _transe_sc(entity_hbm, rel_hbm, head_hbm, relidx_hbm, tail_hbm, out_hbm,
               hidx, ridx, tidx, hrows, rrows, trows, outv, sem):
    wid = lax.axis_index("s") * _NC + lax.axis_index("c")
    base = wid * _BPW

    pltpu.sync_copy(head_hbm.at[pl.ds(base, _BPW)], hidx)
    pltpu.sync_copy(relidx_hbm.at[pl.ds(base, _BPW)], ridx)
    pltpu.sync_copy(tail_hbm.at[pl.ds(base, _BPW)], tidx)

    copies = []
    for j in range(_NCHUNK):
        sl = pl.ds(j * _CHUNK, _CHUNK)
        copies.append(pltpu.async_copy(entity_hbm.at[hidx.at[sl]], hrows.at[sl], sem))
        copies.append(pltpu.async_copy(rel_hbm.at[ridx.at[sl]], rrows.at[sl], sem))
        copies.append(pltpu.async_copy(entity_hbm.at[tidx.at[sl]], trows.at[sl], sem))
    for c in copies:
        c.wait()

    lanes = lax.iota(jnp.int32, _L)

    def group_body(g, carry):
        row = g * _L + lanes

        def d_body(d, acc):
            dd = jnp.full((_L,), 0, jnp.int32) + d
            h = plsc.load_gather(hrows, [row, dd])
            r = plsc.load_gather(rrows, [row, dd])
            t = plsc.load_gather(trows, [row, dd])
            return acc + jnp.abs(h + r - t)

        acc = lax.fori_loop(0, EMBED_DIM, d_body, jnp.zeros((_L,), jnp.float32))
        plsc.store_scatter(outv, [row], acc)
        return carry

    lax.fori_loop(0, _BPW // _L, group_body, 0)

    pltpu.sync_copy(outv, out_hbm.at[pl.ds(base, _BPW)])


def kernel(entity_emb, relation_emb, head, relation, tail):
    return _transe_sc(entity_emb, relation_emb,
                      head.astype(jnp.int32), relation.astype(jnp.int32),
                      tail.astype(jnp.int32))
